# Initial kernel scaffold; baseline (speedup 1.0000x reference)
import jax
import jax.numpy as jnp
from jax.experimental import pallas as pl


def kernel(x, pos, edge_index, params):
    raise NotImplementedError("write your pallas kernel here")


# speedup vs baseline: 3.2578x; 3.2578x over previous
import functools
import math

import jax
import jax.numpy as jnp
import numpy as np
from jax.experimental import pallas as pl

BIG_NEG = -1e9
NEG_HUGE = -1e30
BN_S = float(1.0 / np.sqrt(1.0 + 1e-5))
F32 = jnp.float32
PREC = jax.lax.Precision.HIGHEST


def _iota(shape, dim):
    return jax.lax.broadcasted_iota(jnp.int32, shape, dim)


def _pad_rows(a, rows):
    return jnp.pad(a, ((0, rows - a.shape[0]), (0, 0)))


def _fps_kernel(pos_ref, sel_ref, *, n_real, m_real, m_pad):
    n_pad = pos_ref.shape[1]
    lane_n = _iota((1, n_pad), 1)
    lane_m = _iota((1, m_pad), 1)
    px = pos_ref[0:1, :]
    py = pos_ref[1:2, :]
    pz = pos_ref[2:3, :]

    def extract(j):
        mask = lane_n == j
        return jnp.sum(jnp.where(mask, pos_ref[...], 0.0), axis=1, keepdims=True)

    c0 = extract(jnp.zeros((1, 1), jnp.int32))
    sel0 = jnp.where(lane_m == 0, c0, jnp.zeros((3, m_pad), F32))
    dist0 = jnp.where(lane_n < n_real, jnp.full((1, n_pad), 1e30, F32), -1.0)

    def body(i, state):
        dist, last_c, sel = state
        dx = px - last_c[0:1, 0:1]
        dy = py - last_c[1:2, 0:1]
        dz = pz - last_c[2:3, 0:1]
        d = dx * dx + dy * dy + dz * dz
        dist = jnp.minimum(dist, d)
        mx = jnp.max(dist, axis=1, keepdims=True)
        nxt = jnp.min(jnp.where(dist == mx, lane_n, n_pad), axis=1, keepdims=True)
        c = extract(nxt)
        sel = jnp.where(lane_m == i, c, sel)
        return dist, c, sel

    _, _, sel = jax.lax.fori_loop(1, m_real, body, (dist0, c0, sel0))
    sel_ref[...] = sel


def _fps(pos_t, n_real, m_real, m_pad):
    return pl.pallas_call(
        functools.partial(_fps_kernel, n_real=n_real, m_real=m_real, m_pad=m_pad),
        out_shape=jax.ShapeDtypeStruct((3, m_pad), F32),
    )(pos_t)


def _conv_kernel(sx_ref, q_ref, z_ref, w1_ref, b1_ref, w2_ref, b2_ref,
                 w3_ref, b3_ref, o_ref, *, n_real, r2, topk, cx):
    n_pad = sx_ref.shape[1]
    qb = q_ref.shape[0]
    cin = z_ref.shape[1]
    c3 = o_ref.shape[1]
    sx = sx_ref[0:1, :]
    sy = sx_ref[1:2, :]
    sz = sx_ref[2:3, :]
    qx = q_ref[:, 0:1]
    qy = q_ref[:, 1:2]
    qz = q_ref[:, 2:3]
    dx = qx - sx
    dy = qy - sy
    dz = qz - sz
    d = dx * dx + dy * dy + dz * dz
    lane = _iota((qb, n_pad), 1)
    score = jnp.where((d <= r2) & (lane < n_real), -d, BIG_NEG)
    z_src = z_ref[...]
    col = _iota((qb, cin), 1)
    sub = (jnp.where(col == cx, qx, 0.0) + jnp.where(col == cx + 1, qy, 0.0)
           + jnp.where(col == cx + 2, qz, 0.0))
    w1 = w1_ref[...]
    b1 = b1_ref[...]
    w2 = w2_ref[...]
    b2 = b2_ref[...]
    w3 = w3_ref[...]
    b3 = b3_ref[...]

    def body(k, state):
        score, acc = state
        m = jnp.max(score, axis=1, keepdims=True)
        li = jnp.min(jnp.where(score == m, lane, n_pad), axis=1, keepdims=True)
        ohb = lane == li
        z = jnp.dot(ohb.astype(F32), z_src, preferred_element_type=F32,
                    precision=PREC)
        h = jnp.maximum((jnp.dot(z - sub, w1, preferred_element_type=F32) + b1) * BN_S, 0.0)
        h = jnp.maximum((jnp.dot(h, w2, preferred_element_type=F32) + b2) * BN_S, 0.0)
        h = jnp.dot(h, w3, preferred_element_type=F32) + b3
        valid = m > (BIG_NEG / 2)
        acc = jnp.maximum(acc, jnp.where(valid, h, BIG_NEG))
        score = jnp.where(ohb, BIG_NEG, score)
        return score, acc

    acc0 = jnp.full((qb, c3), BIG_NEG, F32)
    _, acc = jax.lax.fori_loop(0, topk, body, (score, acc0))
    o_ref[...] = jnp.where(acc > (BIG_NEG / 2), acc, 0.0)


def _conv(pos_src_t, pos_q, z_src, w1, b1, w2, b2, w3, b3, *, n_real, r2,
          qb, cx, topk=64):
    n_pad = pos_src_t.shape[1]
    q_pad = pos_q.shape[0]
    cin = z_src.shape[1]
    c1 = w1.shape[1]
    c2 = w2.shape[1]
    c3 = w3.shape[1]
    grid = (q_pad // qb,)
    return pl.pallas_call(
        functools.partial(_conv_kernel, n_real=n_real, r2=r2, topk=topk, cx=cx),
        grid=grid,
        in_specs=[
            pl.BlockSpec((3, n_pad), lambda i: (0, 0)),
            pl.BlockSpec((qb, 3), lambda i: (i, 0)),
            pl.BlockSpec((n_pad, cin), lambda i: (0, 0)),
            pl.BlockSpec((cin, c1), lambda i: (0, 0)),
            pl.BlockSpec((1, c1), lambda i: (0, 0)),
            pl.BlockSpec((c1, c2), lambda i: (0, 0)),
            pl.BlockSpec((1, c2), lambda i: (0, 0)),
            pl.BlockSpec((c2, c3), lambda i: (0, 0)),
            pl.BlockSpec((1, c3), lambda i: (0, 0)),
        ],
        out_specs=pl.BlockSpec((qb, c3), lambda i: (i, 0)),
        out_shape=jax.ShapeDtypeStruct((q_pad, c3), F32),
    )(pos_src_t, pos_q, z_src, w1, b1, w2, b2, w3, b3)


def _sa3fp3_kernel(x2_ref, p2_ref, wa_x_ref, wa_p_ref, ba_ref, wb_ref, bb_ref,
                   wc_ref, bc_ref, wd_x3_ref, wd_x2_ref, bd_ref, we_ref,
                   be_ref, o_ref, *, m_real):
    x2 = x2_ref[...]
    h = jnp.dot(x2, wa_x_ref[...], preferred_element_type=F32)
    h = h + jnp.dot(p2_ref[...], wa_p_ref[...], preferred_element_type=F32)
    h = jnp.maximum((h + ba_ref[...]) * BN_S, 0.0)
    h = jnp.maximum((jnp.dot(h, wb_ref[...], preferred_element_type=F32) + bb_ref[...]) * BN_S, 0.0)
    h = jnp.dot(h, wc_ref[...], preferred_element_type=F32) + bc_ref[...]
    rows = _iota(h.shape, 0)
    h = jnp.where(rows < m_real, h, NEG_HUGE)
    x3 = jnp.max(h, axis=0, keepdims=True)
    g = jnp.dot(x3, wd_x3_ref[...], preferred_element_type=F32)
    g = g + jnp.dot(x2, wd_x2_ref[...], preferred_element_type=F32)
    g = jnp.maximum((g + bd_ref[...]) * BN_S, 0.0)
    g = jnp.dot(g, we_ref[...], preferred_element_type=F32) + be_ref[...]
    o_ref[...] = g


def _sa3fp3(x2, p2, wa_x, wa_p, ba, wb, bb, wc, bc, wd_x3, wd_x2, bd, we, be,
            m_real):
    return pl.pallas_call(
        functools.partial(_sa3fp3_kernel, m_real=m_real),
        out_shape=jax.ShapeDtypeStruct((x2.shape[0], we.shape[1]), F32),
    )(x2, p2, wa_x, wa_p, ba, wb, bb, wc, bc, wd_x3, wd_x2, bd, we, be)


def _knn_mlp_kernel(sx_ref, q_ref, ysrc_ref, xskip_ref, *rest_refs,
                    n_real, acts):
    layer_refs = rest_refs[:-1]
    o_ref = rest_refs[-1]
    n_pad = sx_ref.shape[1]
    qb = q_ref.shape[0]
    sx = sx_ref[0:1, :]
    sy = sx_ref[1:2, :]
    sz = sx_ref[2:3, :]
    qx = q_ref[:, 0:1]
    qy = q_ref[:, 1:2]
    qz = q_ref[:, 2:3]
    dx = qx - sx
    dy = qy - sy
    dz = qz - sz
    d = dx * dx + dy * dy + dz * dz
    lane = _iota((qb, n_pad), 1)
    score = jnp.where(lane < n_real, -d, NEG_HUGE)
    y_src = ysrc_ref[...]
    cs = y_src.shape[1]

    def body(k, state):
        score, num, den = state
        m = jnp.max(score, axis=1, keepdims=True)
        li = jnp.min(jnp.where(score == m, lane, n_pad), axis=1, keepdims=True)
        ohb = lane == li
        w = 1.0 / jnp.maximum(-m, 1e-16)
        y = jnp.dot(ohb.astype(F32), y_src, preferred_element_type=F32, precision=PREC)
        num = num + y * w
        den = den + w
        score = jnp.where(ohb, NEG_HUGE, score)
        return score, num, den

    num0 = jnp.zeros((qb, cs), F32)
    den0 = jnp.zeros((qb, 1), F32)
    _, num, den = jax.lax.fori_loop(0, 3, body, (score, num0, den0))
    h = num / den

    refs = list(layer_refs)
    w_h = refs.pop(0)
    w_skip = refs.pop(0)
    b0 = refs.pop(0)
    h = jnp.dot(h, w_h[...], preferred_element_type=F32)
    h = h + jnp.dot(xskip_ref[...], w_skip[...], preferred_element_type=F32)
    h = h + b0[...]
    n_mm = 1 + len(refs) // 2
    for i in range(n_mm):
        a = acts[i]
        if a == 1:
            h = jnp.maximum(h * BN_S, 0.0)
        elif a == 2:
            h = jnp.maximum(h, 0.0)
        if i + 1 < n_mm:
            w = refs.pop(0)
            b = refs.pop(0)
            h = jnp.dot(h, w[...], preferred_element_type=F32) + b[...]
    o_ref[...] = h


def _knn_mlp(pos_src_t, pos_q, y_src, x_skip, layers, *, n_real, qb, acts):
    n_pad = pos_src_t.shape[1]
    q_pad = pos_q.shape[0]
    cs = y_src.shape[1]
    ck = x_skip.shape[1]
    cout = layers[-2].shape[1]
    grid = (q_pad // qb,)
    const = lambda i: (0, 0)
    in_specs = [
        pl.BlockSpec((3, n_pad), const),
        pl.BlockSpec((qb, 3), lambda i: (i, 0)),
        pl.BlockSpec((n_pad, cs), const),
        pl.BlockSpec((qb, ck), lambda i: (i, 0)),
    ]
    for l in layers:
        in_specs.append(pl.BlockSpec(l.shape, const))
    return pl.pallas_call(
        functools.partial(_knn_mlp_kernel, n_real=n_real, acts=acts),
        grid=grid,
        in_specs=in_specs,
        out_specs=pl.BlockSpec((qb, cout), lambda i: (i, 0)),
        out_shape=jax.ShapeDtypeStruct((q_pad, cout), F32),
    )(pos_src_t, pos_q, y_src, x_skip, *layers)


def _row(b):
    return b.reshape(1, -1)


def kernel(x, pos, edge_index, params):
    n = pos.shape[0]
    m1 = math.ceil(0.2 * n)
    m2 = math.ceil(0.25 * m1)
    m1_pad = ((m1 + 127) // 128) * 128
    m2_pad = ((m2 + 127) // 128) * 128

    pos_t = pos.T

    sel1 = _fps(pos_t, n, m1, m1_pad)
    pos1 = sel1.T
    (w1, b1), (w2, b2), (w3, b3) = params["sa1"]
    z1 = jnp.concatenate([x, pos], axis=1)
    x1 = _conv(pos_t, pos1, z1, w1, _row(b1), w2, _row(b2), w3, _row(b3),
               n_real=n, r2=0.2 * 0.2, qb=128, cx=3)
    rows1 = jnp.arange(m1_pad)[:, None]
    x1 = jnp.where(rows1 < m1, x1, 0.0)

    sel2 = _fps(sel1, m1, m2, m2_pad)
    pos2 = sel2.T
    (w1, b1), (w2, b2), (w3, b3) = params["sa2"]
    z2 = jnp.concatenate([x1, pos1], axis=1)
    x2 = _conv(sel1, pos2, z2, w1, _row(b1), w2, _row(b2), w3,
               _row(b3), n_real=m1, r2=0.4 * 0.4, qb=m2_pad, cx=128)
    rows2 = jnp.arange(m2_pad)[:, None]
    x2 = jnp.where(rows2 < m2, x2, 0.0)

    (wa, ba), (wb, bb), (wc, bc) = params["sa3"]
    (wd, bd), (we, be) = params["fp3"]
    f3 = _sa3fp3(x2, pos2, wa[:256], wa[256:259], _row(ba), wb, _row(bb), wc,
                 _row(bc), wd[:1024], wd[1024:1280], _row(bd), we, _row(be),
                 m2)
    f3 = jnp.where(rows2 < m2, f3, 0.0)

    (wf, bf), (wg, bg) = params["fp2"]
    g2 = _knn_mlp(sel2, pos1, f3, x1,
                  [wf[:256], wf[256:384], _row(bf), wg, _row(bg)],
                  n_real=m2, qb=m1_pad, acts=(1, 0))
    g2 = jnp.where(rows1 < m1, g2, 0.0)

    (wh, bh), (wi, bi), (wj, bj) = params["fp1"]
    (wk, bk), (wl, bl), (wm, bm) = params["head"]
    out = _knn_mlp(sel1, pos, g2, x,
                   [wh[:128], wh[128:131], _row(bh), wi, _row(bi), wj, _row(bj),
                    wk, _row(bk), wl, _row(bl), wm, _row(bm)],
                   n_real=m1, qb=512, acts=(1, 1, 0, 2, 2, 0))
    return out


# speedup vs baseline: 4.5759x; 1.4046x over previous
import functools
import math

import jax
import jax.numpy as jnp
import numpy as np
from jax.experimental import pallas as pl

BIG_NEG = -1e9
NEG_HUGE = -1e30
BN_S = float(1.0 / np.sqrt(1.0 + 1e-5))
F32 = jnp.float32
PREC = jax.lax.Precision.HIGHEST


def _iota(shape, dim):
    return jax.lax.broadcasted_iota(jnp.int32, shape, dim)


def _pad_rows(a, rows):
    return jnp.pad(a, ((0, rows - a.shape[0]), (0, 0)))


def _split3(z):
    zh = z.astype(jnp.bfloat16)
    zr = z - zh.astype(F32)
    zrh = zr.astype(jnp.bfloat16)
    zr2 = (zr - zrh.astype(F32)).astype(jnp.bfloat16)
    return zh, zrh, zr2


def _gather(ohb, parts):
    ohf = ohb.astype(jnp.bfloat16)
    zh, zrh, zr2 = parts
    g = jnp.dot(ohf, zh, preferred_element_type=F32)
    g = g + jnp.dot(ohf, zrh, preferred_element_type=F32)
    g = g + jnp.dot(ohf, zr2, preferred_element_type=F32)
    return g


def _fps_kernel(pos_ref, sel_ref, *, n_real, m_real, m_pad):
    cols = pos_ref.shape[1]
    rows_m = m_pad // 128
    px = pos_ref[0:8, :]
    py = pos_ref[8:16, :]
    pz = pos_ref[16:24, :]
    flat = _iota((8, cols), 0) * cols + _iota((8, cols), 1)
    big = 8 * cols
    row_i = _iota((3 * rows_m, 128), 0)
    col_i = _iota((3 * rows_m, 128), 1)

    def extract(nxt):
        msk = flat == nxt

        def red(p):
            s = jnp.sum(jnp.where(msk, p, 0.0), axis=1, keepdims=True)
            return jnp.sum(s, axis=0, keepdims=True)

        return red(px), red(py), red(pz)

    def scatter(sel, i, cx, cy, cz):
        rb = i // 128
        cb = i % 128
        val = jnp.where(row_i < rows_m, cx,
                        jnp.where(row_i < 2 * rows_m, cy, cz))
        mask = ((row_i % rows_m) == rb) & (col_i == cb)
        return jnp.where(mask, val, sel)

    c0x, c0y, c0z = extract(jnp.zeros((1, 1), jnp.int32))
    sel0 = scatter(jnp.zeros((3 * rows_m, 128), F32), 0, c0x, c0y, c0z)
    dist0 = jnp.where(flat < n_real, jnp.full((8, cols), 1e30, F32), -1.0)

    def body(i, state):
        dist, cx, cy, cz, sel = state
        dx = px - cx
        dy = py - cy
        dz = pz - cz
        d = dx * dx + dy * dy + dz * dz
        dist = jnp.minimum(dist, d)
        mx = jnp.max(jnp.max(dist, axis=1, keepdims=True), axis=0,
                     keepdims=True)
        cand = jnp.where(dist == mx, flat, big)
        nxt = jnp.min(jnp.min(cand, axis=1, keepdims=True), axis=0,
                      keepdims=True)
        ncx, ncy, ncz = extract(nxt)
        sel = scatter(sel, i, ncx, ncy, ncz)
        return dist, ncx, ncy, ncz, sel

    _, _, _, _, sel = jax.lax.fori_loop(1, m_real, body,
                                        (dist0, c0x, c0y, c0z, sel0))
    sel_ref[...] = sel


def _fps(pos_t, n_real, m_real, m_pad):
    n_pad = pos_t.shape[1]
    cols = n_pad // 8
    rows_m = m_pad // 128
    pos_r = pos_t.reshape(3, 8, cols).reshape(24, cols)
    sel_raw = pl.pallas_call(
        functools.partial(_fps_kernel, n_real=n_real, m_real=m_real, m_pad=m_pad),
        out_shape=jax.ShapeDtypeStruct((3 * rows_m, 128), F32),
    )(pos_r)
    return sel_raw.reshape(3, m_pad)


def _conv_kernel(sx_ref, q_ref, z_ref, w1_ref, b1_ref, w2_ref, b2_ref,
                 w3_ref, b3_ref, o_ref, *, n_real, r2, topk, cx):
    n_pad = sx_ref.shape[1]
    qb = q_ref.shape[0]
    cin = z_ref.shape[1]
    c3 = o_ref.shape[1]
    sx = sx_ref[0:1, :]
    sy = sx_ref[1:2, :]
    sz = sx_ref[2:3, :]
    qx = q_ref[:, 0:1]
    qy = q_ref[:, 1:2]
    qz = q_ref[:, 2:3]
    dx = qx - sx
    dy = qy - sy
    dz = qz - sz
    d = dx * dx + dy * dy + dz * dz
    lane = _iota((qb, n_pad), 1)
    score = jnp.where((d <= r2) & (lane < n_real), -d, BIG_NEG)
    z_src = z_ref[...]
    col = _iota((qb, cin), 1)
    sub = (jnp.where(col == cx, qx, 0.0) + jnp.where(col == cx + 1, qy, 0.0)
           + jnp.where(col == cx + 2, qz, 0.0))
    w1 = w1_ref[...]
    b1 = b1_ref[...]
    w2 = w2_ref[...]
    b2 = b2_ref[...]
    w3 = w3_ref[...]
    b3 = b3_ref[...]

    zparts = _split3(z_src)

    def body(k, state):
        m_prev, li_prev, acc = state
        elig = (score < m_prev) | ((score == m_prev) & (lane > li_prev))
        m = jnp.max(jnp.where(elig, score, -jnp.inf), axis=1, keepdims=True)
        li = jnp.min(jnp.where(elig & (score == m), lane, n_pad), axis=1,
                     keepdims=True)
        ohb = lane == li
        z = _gather(ohb, zparts)
        h = jnp.maximum((jnp.dot(z - sub, w1, preferred_element_type=F32) + b1) * BN_S, 0.0)
        h = jnp.maximum((jnp.dot(h, w2, preferred_element_type=F32) + b2) * BN_S, 0.0)
        h = jnp.dot(h, w3, preferred_element_type=F32) + b3
        valid = m > (BIG_NEG / 2)
        acc = jnp.maximum(acc, jnp.where(valid, h, BIG_NEG))
        return m, li, acc

    acc0 = jnp.full((qb, c3), BIG_NEG, F32)
    m0 = jnp.full((qb, 1), jnp.inf, F32)
    li0 = jnp.full((qb, 1), -1, jnp.int32)
    _, _, acc = jax.lax.fori_loop(0, topk, body, (m0, li0, acc0))
    o_ref[...] = jnp.where(acc > (BIG_NEG / 2), acc, 0.0)


def _conv(pos_src_t, pos_q, z_src, w1, b1, w2, b2, w3, b3, *, n_real, r2,
          qb, cx, topk=64):
    n_pad = pos_src_t.shape[1]
    q_pad = pos_q.shape[0]
    cin = z_src.shape[1]
    c1 = w1.shape[1]
    c2 = w2.shape[1]
    c3 = w3.shape[1]
    grid = (q_pad // qb,)
    return pl.pallas_call(
        functools.partial(_conv_kernel, n_real=n_real, r2=r2, topk=topk, cx=cx),
        grid=grid,
        in_specs=[
            pl.BlockSpec((3, n_pad), lambda i: (0, 0)),
            pl.BlockSpec((qb, 3), lambda i: (i, 0)),
            pl.BlockSpec((n_pad, cin), lambda i: (0, 0)),
            pl.BlockSpec((cin, c1), lambda i: (0, 0)),
            pl.BlockSpec((1, c1), lambda i: (0, 0)),
            pl.BlockSpec((c1, c2), lambda i: (0, 0)),
            pl.BlockSpec((1, c2), lambda i: (0, 0)),
            pl.BlockSpec((c2, c3), lambda i: (0, 0)),
            pl.BlockSpec((1, c3), lambda i: (0, 0)),
        ],
        out_specs=pl.BlockSpec((qb, c3), lambda i: (i, 0)),
        out_shape=jax.ShapeDtypeStruct((q_pad, c3), F32),
    )(pos_src_t, pos_q, z_src, w1, b1, w2, b2, w3, b3)


def _sa3fp3_kernel(x2_ref, p2_ref, wa_x_ref, wa_p_ref, ba_ref, wb_ref, bb_ref,
                   wc_ref, bc_ref, wd_x3_ref, wd_x2_ref, bd_ref, we_ref,
                   be_ref, o_ref, *, m_real):
    x2 = x2_ref[...]
    h = jnp.dot(x2, wa_x_ref[...], preferred_element_type=F32)
    h = h + jnp.dot(p2_ref[...], wa_p_ref[...], preferred_element_type=F32)
    h = jnp.maximum((h + ba_ref[...]) * BN_S, 0.0)
    h = jnp.maximum((jnp.dot(h, wb_ref[...], preferred_element_type=F32) + bb_ref[...]) * BN_S, 0.0)
    h = jnp.dot(h, wc_ref[...], preferred_element_type=F32) + bc_ref[...]
    rows = _iota(h.shape, 0)
    h = jnp.where(rows < m_real, h, NEG_HUGE)
    x3 = jnp.max(h, axis=0, keepdims=True)
    g = jnp.dot(x3, wd_x3_ref[...], preferred_element_type=F32)
    g = g + jnp.dot(x2, wd_x2_ref[...], preferred_element_type=F32)
    g = jnp.maximum((g + bd_ref[...]) * BN_S, 0.0)
    g = jnp.dot(g, we_ref[...], preferred_element_type=F32) + be_ref[...]
    o_ref[...] = g


def _sa3fp3(x2, p2, wa_x, wa_p, ba, wb, bb, wc, bc, wd_x3, wd_x2, bd, we, be,
            m_real):
    return pl.pallas_call(
        functools.partial(_sa3fp3_kernel, m_real=m_real),
        out_shape=jax.ShapeDtypeStruct((x2.shape[0], we.shape[1]), F32),
    )(x2, p2, wa_x, wa_p, ba, wb, bb, wc, bc, wd_x3, wd_x2, bd, we, be)


def _knn_mlp_kernel(sx_ref, q_ref, ysrc_ref, xskip_ref, *rest_refs,
                    n_real, acts):
    layer_refs = rest_refs[:-1]
    o_ref = rest_refs[-1]
    n_pad = sx_ref.shape[1]
    qb = q_ref.shape[0]
    sx = sx_ref[0:1, :]
    sy = sx_ref[1:2, :]
    sz = sx_ref[2:3, :]
    qx = q_ref[:, 0:1]
    qy = q_ref[:, 1:2]
    qz = q_ref[:, 2:3]
    dx = qx - sx
    dy = qy - sy
    dz = qz - sz
    d = dx * dx + dy * dy + dz * dz
    lane = _iota((qb, n_pad), 1)
    score = jnp.where(lane < n_real, -d, NEG_HUGE)
    y_src = ysrc_ref[...]
    cs = y_src.shape[1]
    yparts = _split3(y_src)

    def body(k, state):
        m_prev, li_prev, num, den = state
        elig = (score < m_prev) | ((score == m_prev) & (lane > li_prev))
        m = jnp.max(jnp.where(elig, score, -jnp.inf), axis=1, keepdims=True)
        li = jnp.min(jnp.where(elig & (score == m), lane, n_pad), axis=1,
                     keepdims=True)
        ohb = lane == li
        w = 1.0 / jnp.maximum(-m, 1e-16)
        y = _gather(ohb, yparts)
        num = num + y * w
        den = den + w
        return m, li, num, den

    num0 = jnp.zeros((qb, cs), F32)
    den0 = jnp.zeros((qb, 1), F32)
    m0 = jnp.full((qb, 1), jnp.inf, F32)
    li0 = jnp.full((qb, 1), -1, jnp.int32)
    _, _, num, den = jax.lax.fori_loop(0, 3, body, (m0, li0, num0, den0))
    h = num / den

    refs = list(layer_refs)
    w_h = refs.pop(0)
    w_skip = refs.pop(0)
    b0 = refs.pop(0)
    h = jnp.dot(h, w_h[...], preferred_element_type=F32)
    h = h + jnp.dot(xskip_ref[...], w_skip[...], preferred_element_type=F32)
    h = h + b0[...]
    n_mm = 1 + len(refs) // 2
    for i in range(n_mm):
        a = acts[i]
        if a == 1:
            h = jnp.maximum(h * BN_S, 0.0)
        elif a == 2:
            h = jnp.maximum(h, 0.0)
        if i + 1 < n_mm:
            w = refs.pop(0)
            b = refs.pop(0)
            h = jnp.dot(h, w[...], preferred_element_type=F32) + b[...]
    o_ref[...] = h


def _knn_mlp(pos_src_t, pos_q, y_src, x_skip, layers, *, n_real, qb, acts):
    n_pad = pos_src_t.shape[1]
    q_pad = pos_q.shape[0]
    cs = y_src.shape[1]
    ck = x_skip.shape[1]
    cout = layers[-2].shape[1]
    grid = (q_pad // qb,)
    const = lambda i: (0, 0)
    in_specs = [
        pl.BlockSpec((3, n_pad), const),
        pl.BlockSpec((qb, 3), lambda i: (i, 0)),
        pl.BlockSpec((n_pad, cs), const),
        pl.BlockSpec((qb, ck), lambda i: (i, 0)),
    ]
    for l in layers:
        in_specs.append(pl.BlockSpec(l.shape, const))
    return pl.pallas_call(
        functools.partial(_knn_mlp_kernel, n_real=n_real, acts=acts),
        grid=grid,
        in_specs=in_specs,
        out_specs=pl.BlockSpec((qb, cout), lambda i: (i, 0)),
        out_shape=jax.ShapeDtypeStruct((q_pad, cout), F32),
    )(pos_src_t, pos_q, y_src, x_skip, *layers)


def _row(b):
    return b.reshape(1, -1)


def kernel(x, pos, edge_index, params):
    n = pos.shape[0]
    m1 = math.ceil(0.2 * n)
    m2 = math.ceil(0.25 * m1)
    m1_pad = ((m1 + 127) // 128) * 128
    m2_pad = ((m2 + 127) // 128) * 128

    pos_t = pos.T

    sel1 = _fps(pos_t, n, m1, m1_pad)
    pos1 = sel1.T
    (w1, b1), (w2, b2), (w3, b3) = params["sa1"]
    z1 = jnp.concatenate([x, pos], axis=1)
    x1 = _conv(pos_t, pos1, z1, w1, _row(b1), w2, _row(b2), w3, _row(b3),
               n_real=n, r2=0.2 * 0.2, qb=128, cx=3)
    rows1 = jnp.arange(m1_pad)[:, None]
    x1 = jnp.where(rows1 < m1, x1, 0.0)

    sel2 = _fps(sel1, m1, m2, m2_pad)
    pos2 = sel2.T
    (w1, b1), (w2, b2), (w3, b3) = params["sa2"]
    z2 = jnp.concatenate([x1, pos1], axis=1)
    x2 = _conv(sel1, pos2, z2, w1, _row(b1), w2, _row(b2), w3,
               _row(b3), n_real=m1, r2=0.4 * 0.4, qb=m2_pad, cx=128)
    rows2 = jnp.arange(m2_pad)[:, None]
    x2 = jnp.where(rows2 < m2, x2, 0.0)

    (wa, ba), (wb, bb), (wc, bc) = params["sa3"]
    (wd, bd), (we, be) = params["fp3"]
    f3 = _sa3fp3(x2, pos2, wa[:256], wa[256:259], _row(ba), wb, _row(bb), wc,
                 _row(bc), wd[:1024], wd[1024:1280], _row(bd), we, _row(be),
                 m2)
    f3 = jnp.where(rows2 < m2, f3, 0.0)

    (wf, bf), (wg, bg) = params["fp2"]
    g2 = _knn_mlp(sel2, pos1, f3, x1,
                  [wf[:256], wf[256:384], _row(bf), wg, _row(bg)],
                  n_real=m2, qb=m1_pad, acts=(1, 0))
    g2 = jnp.where(rows1 < m1, g2, 0.0)

    (wh, bh), (wi, bi), (wj, bj) = params["fp1"]
    (wk, bk), (wl, bl), (wm, bm) = params["head"]
    out = _knn_mlp(sel1, pos, g2, x,
                   [wh[:128], wh[128:131], _row(bh), wi, _row(bi), wj, _row(bj),
                    wk, _row(bk), wl, _row(bl), wm, _row(bm)],
                   n_real=m1, qb=512, acts=(1, 1, 0, 2, 2, 0))
    return out


# speedup vs baseline: 4.9414x; 1.0799x over previous
import functools
import math

import jax
import jax.numpy as jnp
import numpy as np
from jax.experimental import pallas as pl

BIG_NEG = -1e9
NEG_HUGE = -1e30
BN_S = float(1.0 / np.sqrt(1.0 + 1e-5))
F32 = jnp.float32
PREC = jax.lax.Precision.HIGHEST


def _iota(shape, dim):
    return jax.lax.broadcasted_iota(jnp.int32, shape, dim)


def _pad_rows(a, rows):
    return jnp.pad(a, ((0, rows - a.shape[0]), (0, 0)))


def _split3(z):
    zh = z.astype(jnp.bfloat16)
    zr = z - zh.astype(F32)
    zrh = zr.astype(jnp.bfloat16)
    zr2 = (zr - zrh.astype(F32)).astype(jnp.bfloat16)
    return zh, zrh, zr2


def _gather(ohb, parts):
    ohf = ohb.astype(jnp.bfloat16)
    zh, zrh, zr2 = parts
    g = jnp.dot(ohf, zh, preferred_element_type=F32)
    g = g + jnp.dot(ohf, zrh, preferred_element_type=F32)
    g = g + jnp.dot(ohf, zr2, preferred_element_type=F32)
    return g


def _fps_kernel(pos_ref, sel_ref, *, n_real, m_real, m_pad):
    cols = pos_ref.shape[1]
    rows_m = m_pad // 128
    px = pos_ref[0:8, :]
    py = pos_ref[8:16, :]
    pz = pos_ref[16:24, :]
    flat = _iota((8, cols), 0) * cols + _iota((8, cols), 1)
    big = 8 * cols
    row_i = _iota((3 * rows_m, 128), 0)
    col_i = _iota((3 * rows_m, 128), 1)

    def extract(nxt):
        msk = flat == nxt

        def red(p):
            s = jnp.sum(jnp.where(msk, p, 0.0), axis=1, keepdims=True)
            return jnp.sum(s, axis=0, keepdims=True)

        return red(px), red(py), red(pz)

    def scatter(sel, i, cx, cy, cz):
        rb = i // 128
        cb = i % 128
        val = jnp.where(row_i < rows_m, cx,
                        jnp.where(row_i < 2 * rows_m, cy, cz))
        mask = ((row_i % rows_m) == rb) & (col_i == cb)
        return jnp.where(mask, val, sel)

    c0x, c0y, c0z = extract(jnp.zeros((1, 1), jnp.int32))
    sel0 = scatter(jnp.zeros((3 * rows_m, 128), F32), 0, c0x, c0y, c0z)
    dist0 = jnp.where(flat < n_real, jnp.full((8, cols), 1e30, F32), -1.0)

    def body(i, state):
        dist, cx, cy, cz, sel = state
        dx = px - cx
        dy = py - cy
        dz = pz - cz
        d = dx * dx + dy * dy + dz * dz
        dist = jnp.minimum(dist, d)
        mx = jnp.max(jnp.max(dist, axis=1, keepdims=True), axis=0,
                     keepdims=True)
        cand = jnp.where(dist == mx, flat, big)
        nxt = jnp.min(jnp.min(cand, axis=1, keepdims=True), axis=0,
                      keepdims=True)
        ncx, ncy, ncz = extract(nxt)
        sel = scatter(sel, i, ncx, ncy, ncz)
        return dist, ncx, ncy, ncz, sel

    _, _, _, _, sel = jax.lax.fori_loop(1, m_real, body,
                                        (dist0, c0x, c0y, c0z, sel0))
    sel_ref[...] = sel


def _fps(pos_t, n_real, m_real, m_pad):
    n_pad = pos_t.shape[1]
    cols = n_pad // 8
    rows_m = m_pad // 128
    pos_r = pos_t.reshape(3, 8, cols).reshape(24, cols)
    sel_raw = pl.pallas_call(
        functools.partial(_fps_kernel, n_real=n_real, m_real=m_real, m_pad=m_pad),
        out_shape=jax.ShapeDtypeStruct((3 * rows_m, 128), F32),
    )(pos_r)
    return sel_raw.reshape(3, m_pad)


def _conv_kernel(sx_ref, q_ref, z_ref, w1_ref, b1_ref, w2_ref, b2_ref,
                 w3_ref, b3_ref, o_ref, *, n_real, r2, topk, cx):
    n_pad = sx_ref.shape[1]
    qb = q_ref.shape[0]
    cin = z_ref.shape[1]
    c3 = o_ref.shape[1]
    sx = sx_ref[0:1, :]
    sy = sx_ref[1:2, :]
    sz = sx_ref[2:3, :]
    qx = q_ref[:, 0:1]
    qy = q_ref[:, 1:2]
    qz = q_ref[:, 2:3]
    dx = qx - sx
    dy = qy - sy
    dz = qz - sz
    d = dx * dx + dy * dy + dz * dz
    lane = _iota((qb, n_pad), 1)
    score = jnp.where((d <= r2) & (lane < n_real), -d, BIG_NEG)
    z_src = z_ref[...]
    col = _iota((qb, cin), 1)
    sub = (jnp.where(col == cx, qx, 0.0) + jnp.where(col == cx + 1, qy, 0.0)
           + jnp.where(col == cx + 2, qz, 0.0))
    w1 = w1_ref[...]
    b1 = b1_ref[...]
    w2 = w2_ref[...]
    b2 = b2_ref[...]
    w3 = w3_ref[...]
    b3 = b3_ref[...]

    zparts = _split3(z_src)

    def body(k, state):
        m_prev, li_prev, acc = state
        elig = (score < m_prev) | ((score == m_prev) & (lane > li_prev))
        m = jnp.max(jnp.where(elig, score, -jnp.inf), axis=1, keepdims=True)
        li = jnp.min(jnp.where(elig & (score == m), lane, n_pad), axis=1,
                     keepdims=True)
        ohb = lane == li
        z = _gather(ohb, zparts)
        h = jnp.maximum((jnp.dot(z - sub, w1, preferred_element_type=F32) + b1) * BN_S, 0.0)
        h = jnp.maximum((jnp.dot(h, w2, preferred_element_type=F32) + b2) * BN_S, 0.0)
        h = jnp.dot(h, w3, preferred_element_type=F32) + b3
        valid = m > (BIG_NEG / 2)
        acc = jnp.maximum(acc, jnp.where(valid, h, BIG_NEG))
        return m, li, acc

    acc0 = jnp.full((qb, c3), BIG_NEG, F32)
    m0 = jnp.full((qb, 1), jnp.inf, F32)
    li0 = jnp.full((qb, 1), -1, jnp.int32)
    _, _, acc = jax.lax.fori_loop(0, topk, body, (m0, li0, acc0))
    o_ref[...] = jnp.where(acc > (BIG_NEG / 2), acc, 0.0)


def _conv(pos_src_t, pos_q, z_src, w1, b1, w2, b2, w3, b3, *, n_real, r2,
          qb, cx, topk=64):
    n_pad = pos_src_t.shape[1]
    q_pad = pos_q.shape[0]
    cin = z_src.shape[1]
    c1 = w1.shape[1]
    c2 = w2.shape[1]
    c3 = w3.shape[1]
    grid = (q_pad // qb,)
    return pl.pallas_call(
        functools.partial(_conv_kernel, n_real=n_real, r2=r2, topk=topk, cx=cx),
        grid=grid,
        in_specs=[
            pl.BlockSpec((3, n_pad), lambda i: (0, 0)),
            pl.BlockSpec((qb, 3), lambda i: (i, 0)),
            pl.BlockSpec((n_pad, cin), lambda i: (0, 0)),
            pl.BlockSpec((cin, c1), lambda i: (0, 0)),
            pl.BlockSpec((1, c1), lambda i: (0, 0)),
            pl.BlockSpec((c1, c2), lambda i: (0, 0)),
            pl.BlockSpec((1, c2), lambda i: (0, 0)),
            pl.BlockSpec((c2, c3), lambda i: (0, 0)),
            pl.BlockSpec((1, c3), lambda i: (0, 0)),
        ],
        out_specs=pl.BlockSpec((qb, c3), lambda i: (i, 0)),
        out_shape=jax.ShapeDtypeStruct((q_pad, c3), F32),
    )(pos_src_t, pos_q, z_src, w1, b1, w2, b2, w3, b3)


def _sa3fp3_kernel(x2_ref, p2_ref, wa_x_ref, wa_p_ref, ba_ref, wb_ref, bb_ref,
                   wc_ref, bc_ref, wd_x3_ref, wd_x2_ref, bd_ref, we_ref,
                   be_ref, o_ref, *, m_real):
    x2 = x2_ref[...]
    h = jnp.dot(x2, wa_x_ref[...], preferred_element_type=F32)
    h = h + jnp.dot(p2_ref[...], wa_p_ref[...], preferred_element_type=F32)
    h = jnp.maximum((h + ba_ref[...]) * BN_S, 0.0)
    h = jnp.maximum((jnp.dot(h, wb_ref[...], preferred_element_type=F32) + bb_ref[...]) * BN_S, 0.0)
    h = jnp.dot(h, wc_ref[...], preferred_element_type=F32) + bc_ref[...]
    rows = _iota(h.shape, 0)
    h = jnp.where(rows < m_real, h, NEG_HUGE)
    x3 = jnp.max(h, axis=0, keepdims=True)
    g = jnp.dot(x3, wd_x3_ref[...], preferred_element_type=F32)
    g = g + jnp.dot(x2, wd_x2_ref[...], preferred_element_type=F32)
    g = jnp.maximum((g + bd_ref[...]) * BN_S, 0.0)
    g = jnp.dot(g, we_ref[...], preferred_element_type=F32) + be_ref[...]
    o_ref[...] = g


def _sa3fp3(x2, p2, wa_x, wa_p, ba, wb, bb, wc, bc, wd_x3, wd_x2, bd, we, be,
            m_real):
    return pl.pallas_call(
        functools.partial(_sa3fp3_kernel, m_real=m_real),
        out_shape=jax.ShapeDtypeStruct((x2.shape[0], we.shape[1]), F32),
    )(x2, p2, wa_x, wa_p, ba, wb, bb, wc, bc, wd_x3, wd_x2, bd, we, be)


def _knn_mlp_kernel(sx_ref, q_ref, ysrc_ref, xskip_ref, *rest_refs,
                    n_real, acts):
    layer_refs = rest_refs[:-1]
    o_ref = rest_refs[-1]
    n_pad = sx_ref.shape[1]
    qb = q_ref.shape[0]
    sx = sx_ref[0:1, :]
    sy = sx_ref[1:2, :]
    sz = sx_ref[2:3, :]
    qx = q_ref[:, 0:1]
    qy = q_ref[:, 1:2]
    qz = q_ref[:, 2:3]
    dx = qx - sx
    dy = qy - sy
    dz = qz - sz
    d = dx * dx + dy * dy + dz * dz
    lane = _iota((qb, n_pad), 1)
    score = jnp.where(lane < n_real, -d, NEG_HUGE)
    y_src = ysrc_ref[...]
    cs = y_src.shape[1]
    yparts = _split3(y_src)

    def body(k, state):
        m_prev, li_prev, num, den = state
        elig = (score < m_prev) | ((score == m_prev) & (lane > li_prev))
        m = jnp.max(jnp.where(elig, score, -jnp.inf), axis=1, keepdims=True)
        li = jnp.min(jnp.where(elig & (score == m), lane, n_pad), axis=1,
                     keepdims=True)
        ohb = lane == li
        w = 1.0 / jnp.maximum(-m, 1e-16)
        y = _gather(ohb, yparts)
        num = num + y * w
        den = den + w
        return m, li, num, den

    num0 = jnp.zeros((qb, cs), F32)
    den0 = jnp.zeros((qb, 1), F32)
    m0 = jnp.full((qb, 1), jnp.inf, F32)
    li0 = jnp.full((qb, 1), -1, jnp.int32)
    _, _, num, den = jax.lax.fori_loop(0, 3, body, (m0, li0, num0, den0))
    h = num / den

    refs = list(layer_refs)
    w_h = refs.pop(0)
    w_skip = refs.pop(0)
    b0 = refs.pop(0)
    h = jnp.dot(h, w_h[...], preferred_element_type=F32)
    h = h + jnp.dot(xskip_ref[...], w_skip[...], preferred_element_type=F32)
    h = h + b0[...]
    n_mm = 1 + len(refs) // 2
    for i in range(n_mm):
        a = acts[i]
        if a == 1:
            h = jnp.maximum(h * BN_S, 0.0)
        elif a == 2:
            h = jnp.maximum(h, 0.0)
        if i + 1 < n_mm:
            w = refs.pop(0)
            b = refs.pop(0)
            h = jnp.dot(h, w[...], preferred_element_type=F32) + b[...]
    o_ref[...] = h


def _knn_mlp(pos_src_t, pos_q, y_src, x_skip, layers, *, n_real, qb, acts):
    n_pad = pos_src_t.shape[1]
    q_pad = pos_q.shape[0]
    cs = y_src.shape[1]
    ck = x_skip.shape[1]
    cout = layers[-2].shape[1]
    grid = (q_pad // qb,)
    const = lambda i: (0, 0)
    in_specs = [
        pl.BlockSpec((3, n_pad), const),
        pl.BlockSpec((qb, 3), lambda i: (i, 0)),
        pl.BlockSpec((n_pad, cs), const),
        pl.BlockSpec((qb, ck), lambda i: (i, 0)),
    ]
    for l in layers:
        in_specs.append(pl.BlockSpec(l.shape, const))
    return pl.pallas_call(
        functools.partial(_knn_mlp_kernel, n_real=n_real, acts=acts),
        grid=grid,
        in_specs=in_specs,
        out_specs=pl.BlockSpec((qb, cout), lambda i: (i, 0)),
        out_shape=jax.ShapeDtypeStruct((q_pad, cout), F32),
    )(pos_src_t, pos_q, y_src, x_skip, *layers)


def _row(b):
    return b.reshape(1, -1)


def kernel(x, pos, edge_index, params):
    n = pos.shape[0]
    m1 = math.ceil(0.2 * n)
    m2 = math.ceil(0.25 * m1)
    m1_pad = ((m1 + 127) // 128) * 128
    m2_pad = ((m2 + 127) // 128) * 128

    pos_t = pos.T

    sel1 = _fps(pos_t, n, m1, m1_pad)
    pos1 = sel1.T
    (w1, b1), (w2, b2), (w3, b3) = params["sa1"]
    z1 = jnp.concatenate([x, pos], axis=1)
    x1 = _conv(pos_t, pos1, z1, w1, _row(b1), w2, _row(b2), w3, _row(b3),
               n_real=n, r2=0.2 * 0.2, qb=416, cx=3)
    rows1 = jnp.arange(m1_pad)[:, None]
    x1 = jnp.where(rows1 < m1, x1, 0.0)

    sel2 = _fps(sel1, m1, m2, m2_pad)
    pos2 = sel2.T
    (w1, b1), (w2, b2), (w3, b3) = params["sa2"]
    z2 = jnp.concatenate([x1, pos1], axis=1)
    x2 = _conv(sel1, pos2, z2, w1, _row(b1), w2, _row(b2), w3,
               _row(b3), n_real=m1, r2=0.4 * 0.4, qb=m2_pad, cx=128)
    rows2 = jnp.arange(m2_pad)[:, None]
    x2 = jnp.where(rows2 < m2, x2, 0.0)

    (wa, ba), (wb, bb), (wc, bc) = params["sa3"]
    (wd, bd), (we, be) = params["fp3"]
    f3 = _sa3fp3(x2, pos2, wa[:256], wa[256:259], _row(ba), wb, _row(bb), wc,
                 _row(bc), wd[:1024], wd[1024:1280], _row(bd), we, _row(be),
                 m2)
    f3 = jnp.where(rows2 < m2, f3, 0.0)

    (wf, bf), (wg, bg) = params["fp2"]
    g2 = _knn_mlp(sel2, pos1, f3, x1,
                  [wf[:256], wf[256:384], _row(bf), wg, _row(bg)],
                  n_real=m2, qb=m1_pad, acts=(1, 0))
    g2 = jnp.where(rows1 < m1, g2, 0.0)

    (wh, bh), (wi, bi), (wj, bj) = params["fp1"]
    (wk, bk), (wl, bl), (wm, bm) = params["head"]
    out = _knn_mlp(sel1, pos, g2, x,
                   [wh[:128], wh[128:131], _row(bh), wi, _row(bi), wj, _row(bj),
                    wk, _row(bk), wl, _row(bl), wm, _row(bm)],
                   n_real=m1, qb=512, acts=(1, 1, 0, 2, 2, 0))
    return out


# speedup vs baseline: 4.9509x; 1.0019x over previous
import functools
import math

import jax
import jax.numpy as jnp
import numpy as np
from jax.experimental import pallas as pl

BIG_NEG = -1e9
NEG_HUGE = -1e30
BN_S = float(1.0 / np.sqrt(1.0 + 1e-5))
F32 = jnp.float32
PREC = jax.lax.Precision.HIGHEST


def _iota(shape, dim):
    return jax.lax.broadcasted_iota(jnp.int32, shape, dim)


def _pad_rows(a, rows):
    return jnp.pad(a, ((0, rows - a.shape[0]), (0, 0)))


def _split3(z):
    zh = z.astype(jnp.bfloat16)
    zr = z - zh.astype(F32)
    zrh = zr.astype(jnp.bfloat16)
    zr2 = (zr - zrh.astype(F32)).astype(jnp.bfloat16)
    return zh, zrh, zr2


def _gather(ohb, parts):
    ohf = ohb.astype(jnp.bfloat16)
    zh, zrh, zr2 = parts
    g = jnp.dot(ohf, zh, preferred_element_type=F32)
    g = g + jnp.dot(ohf, zrh, preferred_element_type=F32)
    g = g + jnp.dot(ohf, zr2, preferred_element_type=F32)
    return g


def _fps_kernel(pos_ref, sel_ref, *, n_real, m_real, m_pad):
    cols = pos_ref.shape[1]
    rows_m = m_pad // 128
    px = pos_ref[0:8, :]
    py = pos_ref[8:16, :]
    pz = pos_ref[16:24, :]
    flat = _iota((8, cols), 0) * cols + _iota((8, cols), 1)
    big = 8 * cols
    row_i = _iota((3 * rows_m, 128), 0)
    col_i = _iota((3 * rows_m, 128), 1)

    def extract(nxt):
        msk = flat == nxt

        def red(p):
            s = jnp.sum(jnp.where(msk, p, 0.0), axis=1, keepdims=True)
            return jnp.sum(s, axis=0, keepdims=True)

        return red(px), red(py), red(pz)

    def scatter(sel, i, cx, cy, cz):
        rb = i // 128
        cb = i % 128
        val = jnp.where(row_i < rows_m, cx,
                        jnp.where(row_i < 2 * rows_m, cy, cz))
        mask = ((row_i % rows_m) == rb) & (col_i == cb)
        return jnp.where(mask, val, sel)

    c0x, c0y, c0z = extract(jnp.zeros((1, 1), jnp.int32))
    sel0 = scatter(jnp.zeros((3 * rows_m, 128), F32), 0, c0x, c0y, c0z)
    dist0 = jnp.where(flat < n_real, jnp.full((8, cols), 1e30, F32), -1.0)

    def body(i, state):
        dist, cx, cy, cz, sel = state
        dx = px - cx
        dy = py - cy
        dz = pz - cz
        d = dx * dx + dy * dy + dz * dz
        dist = jnp.minimum(dist, d)
        mx = jnp.max(jnp.max(dist, axis=1, keepdims=True), axis=0,
                     keepdims=True)
        cand = jnp.where(dist == mx, flat, big)
        nxt = jnp.min(jnp.min(cand, axis=1, keepdims=True), axis=0,
                      keepdims=True)
        ncx, ncy, ncz = extract(nxt)
        sel = scatter(sel, i, ncx, ncy, ncz)
        return dist, ncx, ncy, ncz, sel

    _, _, _, _, sel = jax.lax.fori_loop(1, m_real, body,
                                        (dist0, c0x, c0y, c0z, sel0))
    sel_ref[...] = sel


def _fps(pos_t, n_real, m_real, m_pad):
    n_pad = pos_t.shape[1]
    cols = n_pad // 8
    rows_m = m_pad // 128
    pos_r = pos_t.reshape(3, 8, cols).reshape(24, cols)
    sel_raw = pl.pallas_call(
        functools.partial(_fps_kernel, n_real=n_real, m_real=m_real, m_pad=m_pad),
        out_shape=jax.ShapeDtypeStruct((3 * rows_m, 128), F32),
    )(pos_r)
    return sel_raw.reshape(3, m_pad)


def _conv_kernel(sx_ref, q_ref, z_ref, w1_ref, b1_ref, w2_ref, b2_ref,
                 w3_ref, b3_ref, o_ref, *, n_real, r2, topk, cx):
    n_pad = sx_ref.shape[1]
    qb = q_ref.shape[0]
    cin = z_ref.shape[1]
    c3 = o_ref.shape[1]
    sx = sx_ref[0:1, :]
    sy = sx_ref[1:2, :]
    sz = sx_ref[2:3, :]
    qx = q_ref[:, 0:1]
    qy = q_ref[:, 1:2]
    qz = q_ref[:, 2:3]
    dx = qx - sx
    dy = qy - sy
    dz = qz - sz
    d = dx * dx + dy * dy + dz * dz
    lane = _iota((qb, n_pad), 1)
    score = jnp.where((d <= r2) & (lane < n_real), -d, BIG_NEG)
    z_src = z_ref[...]
    col = _iota((qb, cin), 1)
    sub = (jnp.where(col == cx, qx, 0.0) + jnp.where(col == cx + 1, qy, 0.0)
           + jnp.where(col == cx + 2, qz, 0.0))
    w1 = w1_ref[...]
    b1 = b1_ref[...]
    w2 = w2_ref[...]
    b2 = b2_ref[...]
    w3 = w3_ref[...]
    b3 = b3_ref[...]

    zparts = _split3(z_src)

    def body(k, state):
        m_prev, li_prev, acc = state
        elig = (score < m_prev) | ((score == m_prev) & (lane > li_prev))
        m = jnp.max(jnp.where(elig, score, -jnp.inf), axis=1, keepdims=True)
        li = jnp.min(jnp.where(elig & (score == m), lane, n_pad), axis=1,
                     keepdims=True)
        ohb = lane == li
        z = _gather(ohb, zparts)
        h = jnp.maximum((jnp.dot(z - sub, w1, preferred_element_type=F32) + b1) * BN_S, 0.0)
        h = jnp.maximum((jnp.dot(h, w2, preferred_element_type=F32) + b2) * BN_S, 0.0)
        h = jnp.dot(h, w3, preferred_element_type=F32) + b3
        valid = m > (BIG_NEG / 2)
        acc = jnp.maximum(acc, jnp.where(valid, h, BIG_NEG))
        return m, li, acc

    acc0 = jnp.full((qb, c3), BIG_NEG, F32)
    m0 = jnp.full((qb, 1), jnp.inf, F32)
    li0 = jnp.full((qb, 1), -1, jnp.int32)
    _, _, acc = jax.lax.fori_loop(0, topk, body, (m0, li0, acc0))
    o_ref[...] = jnp.where(acc > (BIG_NEG / 2), acc, 0.0)


def _conv(pos_src_t, pos_q, z_src, w1, b1, w2, b2, w3, b3, *, n_real, r2,
          qb, cx, topk=64):
    n_pad = pos_src_t.shape[1]
    q_pad = pos_q.shape[0]
    cin = z_src.shape[1]
    c1 = w1.shape[1]
    c2 = w2.shape[1]
    c3 = w3.shape[1]
    grid = (q_pad // qb,)
    return pl.pallas_call(
        functools.partial(_conv_kernel, n_real=n_real, r2=r2, topk=topk, cx=cx),
        grid=grid,
        in_specs=[
            pl.BlockSpec((3, n_pad), lambda i: (0, 0)),
            pl.BlockSpec((qb, 3), lambda i: (i, 0)),
            pl.BlockSpec((n_pad, cin), lambda i: (0, 0)),
            pl.BlockSpec((cin, c1), lambda i: (0, 0)),
            pl.BlockSpec((1, c1), lambda i: (0, 0)),
            pl.BlockSpec((c1, c2), lambda i: (0, 0)),
            pl.BlockSpec((1, c2), lambda i: (0, 0)),
            pl.BlockSpec((c2, c3), lambda i: (0, 0)),
            pl.BlockSpec((1, c3), lambda i: (0, 0)),
        ],
        out_specs=pl.BlockSpec((qb, c3), lambda i: (i, 0)),
        out_shape=jax.ShapeDtypeStruct((q_pad, c3), F32),
    )(pos_src_t, pos_q, z_src, w1, b1, w2, b2, w3, b3)


def _sa3fp3_kernel(x2_ref, p2_ref, wa_x_ref, wa_p_ref, ba_ref, wb_ref, bb_ref,
                   wc_ref, bc_ref, wd_x3_ref, wd_x2_ref, bd_ref, we_ref,
                   be_ref, o_ref, *, m_real):
    x2 = x2_ref[...]
    h = jnp.dot(x2, wa_x_ref[...], preferred_element_type=F32)
    h = h + jnp.dot(p2_ref[...], wa_p_ref[...], preferred_element_type=F32)
    h = jnp.maximum((h + ba_ref[...]) * BN_S, 0.0)
    h = jnp.maximum((jnp.dot(h, wb_ref[...], preferred_element_type=F32) + bb_ref[...]) * BN_S, 0.0)
    h = jnp.dot(h, wc_ref[...], preferred_element_type=F32) + bc_ref[...]
    rows = _iota(h.shape, 0)
    h = jnp.where(rows < m_real, h, NEG_HUGE)
    x3 = jnp.max(h, axis=0, keepdims=True)
    g = jnp.dot(x3, wd_x3_ref[...], preferred_element_type=F32)
    g = g + jnp.dot(x2, wd_x2_ref[...], preferred_element_type=F32)
    g = jnp.maximum((g + bd_ref[...]) * BN_S, 0.0)
    g = jnp.dot(g, we_ref[...], preferred_element_type=F32) + be_ref[...]
    o_ref[...] = g


def _sa3fp3(x2, p2, wa_x, wa_p, ba, wb, bb, wc, bc, wd_x3, wd_x2, bd, we, be,
            m_real):
    return pl.pallas_call(
        functools.partial(_sa3fp3_kernel, m_real=m_real),
        out_shape=jax.ShapeDtypeStruct((x2.shape[0], we.shape[1]), F32),
    )(x2, p2, wa_x, wa_p, ba, wb, bb, wc, bc, wd_x3, wd_x2, bd, we, be)


def _knn_mlp_kernel(sx_ref, q_ref, ysrc_ref, xskip_ref, *rest_refs,
                    n_real, acts):
    layer_refs = rest_refs[:-1]
    o_ref = rest_refs[-1]
    n_pad = sx_ref.shape[1]
    qb = q_ref.shape[0]
    sx = sx_ref[0:1, :]
    sy = sx_ref[1:2, :]
    sz = sx_ref[2:3, :]
    qx = q_ref[:, 0:1]
    qy = q_ref[:, 1:2]
    qz = q_ref[:, 2:3]
    dx = qx - sx
    dy = qy - sy
    dz = qz - sz
    d = dx * dx + dy * dy + dz * dz
    lane = _iota((qb, n_pad), 1)
    score = jnp.where(lane < n_real, -d, NEG_HUGE)
    y_src = ysrc_ref[...]
    cs = y_src.shape[1]
    yparts = _split3(y_src)

    def body(k, state):
        m_prev, li_prev, num, den = state
        elig = (score < m_prev) | ((score == m_prev) & (lane > li_prev))
        m = jnp.max(jnp.where(elig, score, -jnp.inf), axis=1, keepdims=True)
        li = jnp.min(jnp.where(elig & (score == m), lane, n_pad), axis=1,
                     keepdims=True)
        ohb = lane == li
        w = 1.0 / jnp.maximum(-m, 1e-16)
        y = _gather(ohb, yparts)
        num = num + y * w
        den = den + w
        return m, li, num, den

    num0 = jnp.zeros((qb, cs), F32)
    den0 = jnp.zeros((qb, 1), F32)
    m0 = jnp.full((qb, 1), jnp.inf, F32)
    li0 = jnp.full((qb, 1), -1, jnp.int32)
    _, _, num, den = jax.lax.fori_loop(0, 3, body, (m0, li0, num0, den0))
    h = num / den

    refs = list(layer_refs)
    w_h = refs.pop(0)
    w_skip = refs.pop(0)
    b0 = refs.pop(0)
    h = jnp.dot(h, w_h[...], preferred_element_type=F32)
    h = h + jnp.dot(xskip_ref[...], w_skip[...], preferred_element_type=F32)
    h = h + b0[...]
    n_mm = 1 + len(refs) // 2
    for i in range(n_mm):
        a = acts[i]
        if a == 1:
            h = jnp.maximum(h * BN_S, 0.0)
        elif a == 2:
            h = jnp.maximum(h, 0.0)
        if i + 1 < n_mm:
            w = refs.pop(0)
            b = refs.pop(0)
            h = jnp.dot(h, w[...], preferred_element_type=F32) + b[...]
    o_ref[...] = h


def _knn_mlp(pos_src_t, pos_q, y_src, x_skip, layers, *, n_real, qb, acts):
    n_pad = pos_src_t.shape[1]
    q_pad = pos_q.shape[0]
    cs = y_src.shape[1]
    ck = x_skip.shape[1]
    cout = layers[-2].shape[1]
    grid = (q_pad // qb,)
    const = lambda i: (0, 0)
    in_specs = [
        pl.BlockSpec((3, n_pad), const),
        pl.BlockSpec((qb, 3), lambda i: (i, 0)),
        pl.BlockSpec((n_pad, cs), const),
        pl.BlockSpec((qb, ck), lambda i: (i, 0)),
    ]
    for l in layers:
        in_specs.append(pl.BlockSpec(l.shape, const))
    return pl.pallas_call(
        functools.partial(_knn_mlp_kernel, n_real=n_real, acts=acts),
        grid=grid,
        in_specs=in_specs,
        out_specs=pl.BlockSpec((qb, cout), lambda i: (i, 0)),
        out_shape=jax.ShapeDtypeStruct((q_pad, cout), F32),
    )(pos_src_t, pos_q, y_src, x_skip, *layers)


def _row(b):
    return b.reshape(1, -1)


def kernel(x, pos, edge_index, params):
    n = pos.shape[0]
    m1 = math.ceil(0.2 * n)
    m2 = math.ceil(0.25 * m1)
    m1_pad = ((m1 + 127) // 128) * 128
    m2_pad = ((m2 + 127) // 128) * 128

    pos_t = pos.T

    sel1 = _fps(pos_t, n, m1, m1_pad)
    pos1 = sel1.T
    (w1, b1), (w2, b2), (w3, b3) = params["sa1"]
    z1 = jnp.concatenate([x, pos], axis=1)
    x1 = _conv(pos_t, pos1, z1, w1, _row(b1), w2, _row(b2), w3, _row(b3),
               n_real=n, r2=0.2 * 0.2, qb=416, cx=3)
    rows1 = jnp.arange(m1_pad)[:, None]
    x1 = jnp.where(rows1 < m1, x1, 0.0)

    sel2 = _fps(sel1, m1, m2, m2_pad)
    pos2 = sel2.T
    (w1, b1), (w2, b2), (w3, b3) = params["sa2"]
    z2 = jnp.concatenate([x1, pos1], axis=1)
    x2 = _conv(sel1, pos2, z2, w1, _row(b1), w2, _row(b2), w3,
               _row(b3), n_real=m1, r2=0.4 * 0.4, qb=m2_pad, cx=128)
    rows2 = jnp.arange(m2_pad)[:, None]
    x2 = jnp.where(rows2 < m2, x2, 0.0)

    (wa, ba), (wb, bb), (wc, bc) = params["sa3"]
    (wd, bd), (we, be) = params["fp3"]
    f3 = _sa3fp3(x2, pos2, wa[:256], wa[256:259], _row(ba), wb, _row(bb), wc,
                 _row(bc), wd[:1024], wd[1024:1280], _row(bd), we, _row(be),
                 m2)
    f3 = jnp.where(rows2 < m2, f3, 0.0)

    (wf, bf), (wg, bg) = params["fp2"]
    g2 = _knn_mlp(sel2, pos1, f3, x1,
                  [wf[:256], wf[256:384], _row(bf), wg, _row(bg)],
                  n_real=m2, qb=m1_pad, acts=(1, 0))
    g2 = jnp.where(rows1 < m1, g2, 0.0)

    (wh, bh), (wi, bi), (wj, bj) = params["fp1"]
    (wk, bk), (wl, bl), (wm, bm) = params["head"]
    out = _knn_mlp(sel1, pos, g2, x,
                   [wh[:128], wh[128:131], _row(bh), wi, _row(bi), wj, _row(bj),
                    wk, _row(bk), wl, _row(bl), wm, _row(bm)],
                   n_real=m1, qb=1024, acts=(1, 1, 0, 2, 2, 0))
    return out
